# baseline (speedup 1.0000x reference)
import functools

import jax
import jax.numpy as jnp
from jax import lax
from jax.experimental import pallas as pl
from jax.experimental.pallas import tpu as pltpu
from jax.experimental.pallas import tpu_sc as plsc

D_MODEL = 64
NUM_CORES = 2
NUM_SUBCORES = 16
NW = NUM_CORES * NUM_SUBCORES
LANES = 16
BBLK = 128
H = 50
NBUF = 3


def _gather_kernel(
    x_hbm, table_hbm, out_hbm, xr_v, idx_v, rows_v, tile_v, table_sh,
    gsem, osem, tsem
):
    wid = lax.axis_index("s") * NUM_CORES + lax.axis_index("c")

    @pl.when(lax.axis_index("s") == 0)
    def _():
        pltpu.async_copy(table_hbm, table_sh, tsem)

    pltpu.sync_copy(x_hbm.at[pl.ds(BBLK * wid, BBLK)], xr_v)
    lanes16 = lax.iota(jnp.int32, LANES)

    @plsc.parallel_loop(0, H, unroll=5)
    def xbody(h):
        hs = jnp.full((LANES,), h, jnp.int32)
        for k in range(8):
            vec = plsc.load_gather(xr_v, [lanes16 + 16 * k, hs])
            idx_v.at[h][pl.ds(16 * k, LANES)] = vec

    @pl.when(lax.axis_index("s") == 0)
    def _():
        pltpu.make_async_copy(table_hbm, table_sh, tsem).wait()

    plsc.subcore_barrier()

    def gather_desc(h, b):
        return pltpu.make_async_copy(
            table_sh.at[idx_v.at[h]], rows_v.at[b], gsem
        )

    def store_desc(h, b):
        return pltpu.make_async_copy(
            tile_v.at[b, :, :, pl.ds(0, BBLK)], out_hbm.at[h, :, wid], osem
        )

    lanes = lax.iota(jnp.int32, LANES)
    rvecs = [(lanes + 16 * m) // 8 for m in range(4)]
    dvecs = [(lanes + 16 * m) % 8 for m in range(4)]

    gather_desc(0, 0).start()
    gather_desc(1, 1).start()

    def body(h, carry):
        b = h % NBUF
        gather_desc(h, b).wait()

        @pl.when(h + 2 < H)
        def _():
            gather_desc(h + 2, (h + 2) % NBUF).start()

        @pl.when(h >= NBUF)
        def _():
            store_desc(h - NBUF, b).wait()

        @plsc.parallel_loop(0, BBLK, unroll=8)
        def tbody(bp):
            bs = jnp.full((LANES,), bp, jnp.int32)
            for m in range(4):
                vec = rows_v[b, bp, pl.ds(16 * m, LANES)]
                plsc.store_scatter(tile_v.at[b], [rvecs[m], dvecs[m], bs], vec)

        store_desc(h, b).start()
        return carry

    lax.fori_loop(0, H, body, 0)
    store_desc(H - 3, (H - 3) % NBUF).wait()
    store_desc(H - 2, (H - 2) % NBUF).wait()
    store_desc(H - 1, (H - 1) % NBUF).wait()


@jax.jit
def _run(x, table):
    mesh = plsc.VectorSubcoreMesh(core_axis_name="c", subcore_axis_name="s")
    k = functools.partial(
        pl.kernel,
        mesh=mesh,
        compiler_params=pltpu.CompilerParams(
            use_tc_tiling_on_sc=False, needs_layout_passes=False
        ),
        out_type=jax.ShapeDtypeStruct((H, 8, NW, 8, BBLK), jnp.float32),
        scratch_types=[
            pltpu.VMEM((BBLK, H), jnp.int32),
            pltpu.VMEM((H, BBLK), jnp.int32),
            pltpu.VMEM((NBUF, BBLK, D_MODEL), jnp.float32),
            pltpu.VMEM((NBUF, 8, 8, BBLK + 1), jnp.float32),
            pltpu.VMEM_SHARED((5001, D_MODEL), jnp.float32),
            pltpu.SemaphoreType.DMA,
            pltpu.SemaphoreType.DMA,
            pltpu.SemaphoreType.DMA,
        ],
    )(_gather_kernel)
    return k(x, table)


def kernel(x, se):
    bsz, h = x.shape
    table = se.reshape(se.shape[0], D_MODEL)
    out5 = _run(x, table)
    out = out5.transpose(2, 4, 0, 1, 3).reshape(bsz, h, D_MODEL)
    return out[:, :, None, :]


# speedup vs baseline: 1.0432x; 1.0432x over previous
import functools

import jax
import jax.numpy as jnp
from jax import lax
from jax.experimental import pallas as pl
from jax.experimental.pallas import tpu as pltpu
from jax.experimental.pallas import tpu_sc as plsc

D_MODEL = 64
NUM_CORES = 2
NUM_SUBCORES = 16
NW = NUM_CORES * NUM_SUBCORES
LANES = 16
BBLK = 128
H = 50
NBUF = 3


def _gather_kernel(
    xp_hbm, table_hbm, out_hbm, idx_v, rows_v, tile_v, table_sh, gsem, osem, tsem
):
    wid = lax.axis_index("s") * NUM_CORES + lax.axis_index("c")

    @pl.when(lax.axis_index("s") == 0)
    def _():
        pltpu.async_copy(table_hbm, table_sh, tsem)

    pltpu.sync_copy(xp_hbm.at[wid], idx_v)

    def gather_hbm_desc(h, b):
        return pltpu.make_async_copy(
            table_hbm.at[idx_v.at[h]], rows_v.at[b], gsem
        )

    def gather_desc(h, b):
        return pltpu.make_async_copy(
            table_sh.at[idx_v.at[h]], rows_v.at[b], gsem
        )

    def store_desc(h, b):
        return pltpu.make_async_copy(
            tile_v.at[b, :, :, pl.ds(0, BBLK)], out_hbm.at[h, :, wid], osem
        )

    lanes = lax.iota(jnp.int32, LANES)
    rvecs = [(lanes + 16 * m) // 8 for m in range(4)]
    dvecs = [(lanes + 16 * m) % 8 for m in range(4)]

    gather_hbm_desc(0, 0).start()
    gather_hbm_desc(1, 1).start()

    @pl.when(lax.axis_index("s") == 0)
    def _():
        pltpu.make_async_copy(table_hbm, table_sh, tsem).wait()

    plsc.subcore_barrier()

    def body(h, carry):
        b = h % NBUF
        gather_desc(h, b).wait()

        @pl.when(h + 2 < H)
        def _():
            gather_desc(h + 2, (h + 2) % NBUF).start()

        @pl.when(h >= NBUF)
        def _():
            store_desc(h - NBUF, b).wait()

        @plsc.parallel_loop(0, BBLK, unroll=8)
        def tbody(bp):
            bs = jnp.full((LANES,), bp, jnp.int32)
            for m in range(4):
                vec = rows_v[b, bp, pl.ds(16 * m, LANES)]
                plsc.store_scatter(tile_v.at[b], [rvecs[m], dvecs[m], bs], vec)

        store_desc(h, b).start()
        return carry

    lax.fori_loop(0, H, body, 0)
    store_desc(H - 3, (H - 3) % NBUF).wait()
    store_desc(H - 2, (H - 2) % NBUF).wait()
    store_desc(H - 1, (H - 1) % NBUF).wait()


@jax.jit
def _run(xp, table):
    mesh = plsc.VectorSubcoreMesh(core_axis_name="c", subcore_axis_name="s")
    k = functools.partial(
        pl.kernel,
        mesh=mesh,
        compiler_params=pltpu.CompilerParams(
            use_tc_tiling_on_sc=False, needs_layout_passes=False
        ),
        out_type=jax.ShapeDtypeStruct((H, 8, NW, 8, BBLK), jnp.float32),
        scratch_types=[
            pltpu.VMEM((H, BBLK), jnp.int32),
            pltpu.VMEM((NBUF, BBLK, D_MODEL), jnp.float32),
            pltpu.VMEM((NBUF, 8, 8, BBLK + 1), jnp.float32),
            pltpu.VMEM_SHARED((5001, D_MODEL), jnp.float32),
            pltpu.SemaphoreType.DMA,
            pltpu.SemaphoreType.DMA,
            pltpu.SemaphoreType.DMA,
        ],
    )(_gather_kernel)
    return k(xp, table)


def kernel(x, se):
    bsz, h = x.shape
    table = se.reshape(se.shape[0], D_MODEL)
    xp = x.reshape(NW, BBLK, h).transpose(0, 2, 1)
    out5 = _run(xp, table)
    out = out5.transpose(2, 4, 0, 1, 3).reshape(bsz, h, D_MODEL)
    return out[:, :, None, :]
